# baseline (speedup 1.0000x reference)
import jax
import jax.numpy as jnp
from jax.experimental import pallas as pl
from jax.experimental.pallas import tpu as pltpu

B = 64
E = 300
H = 256
TC = 637
TT = 24
TP = 640
C = 16
NC = TP // C
PROJ_BM = 2048
N_ROWS = TP * B
N_REAL = TC * B


def _sigmoid(x):
    return jax.nn.sigmoid(x)


def _proj_kernel(x_ref, w_ref, b_ref, of_ref, ob_ref):
    x = x_ref[...].astype(jnp.bfloat16)
    rf = jnp.dot(x, w_ref[0], preferred_element_type=jnp.float32)
    of_ref[...] = (rf + b_ref[0]).astype(jnp.bfloat16).reshape(
        PROJ_BM // B, B, 4 * H)
    rb = jnp.dot(x, w_ref[1], preferred_element_type=jnp.float32)
    ob_ref[...] = (rb + b_ref[1]).astype(jnp.bfloat16).reshape(
        PROJ_BM // B, B, 4 * H)


def _proj2_kernel(x_ref, pfa_ref, pba_ref, w_ref, b_ref, of_ref, ob_ref):
    del pfa_ref, pba_ref
    _proj_kernel(x_ref, w_ref, b_ref, of_ref, ob_ref)


def _lstm_step(p, h, c, whh_t):
    g = p + jnp.dot(h.astype(whh_t.dtype), whh_t,
                    preferred_element_type=jnp.float32)
    i = g[:, :H]
    f = g[:, H:2 * H]
    gg = g[:, 2 * H:3 * H]
    o = g[:, 3 * H:]
    c_new = _sigmoid(f) * c + _sigmoid(i) * jnp.tanh(gg)
    h_new = _sigmoid(o) * jnp.tanh(c_new)
    return h_new, c_new


def _title_kernel(tx_ref, tw_ref, tb_ref, whf_ref, whb_ref, attw_ref, q_ref,
                  pre_s, h_s, c_s):
    tx = tx_ref[...]
    pf = jnp.dot(tx, tw_ref[:, :4 * H], preferred_element_type=jnp.float32)
    pre_s[...] = (pf + tb_ref[:, :4 * H]).reshape(TT, B, 4 * H)
    h_s[...] = jnp.zeros((B, H), jnp.float32)
    c_s[...] = jnp.zeros((B, H), jnp.float32)

    def body(j, _):
        h, c = _lstm_step(pre_s[j], h_s[...], c_s[...], whf_ref[...])
        h_s[...] = h
        c_s[...] = c
        return ()

    jax.lax.fori_loop(0, TT, body, ())
    pb = jnp.dot(tx[(TT - 1) * B:, :], tw_ref[:, 4 * H:],
                 preferred_element_type=jnp.float32) + tb_ref[:, 4 * H:]
    zero = jnp.zeros((B, H), jnp.float32)
    hb, _ = _lstm_step(pb, zero, zero, whb_ref[...])
    title_last = jnp.concatenate([h_s[...], hb], axis=1)
    q_ref[...] = jnp.dot(title_last, attw_ref[...],
                         preferred_element_type=jnp.float32)


def _scan_kernel(pf_ref, pb_ref, whh_ref, q_ref, sf_ref, sb_ref,
                 hf_s, cf_s, hb_s, cb_s, Hf_s, Hb_s):
    tc = pl.program_id(0)

    @pl.when(tc == 0)
    def _():
        z = jnp.zeros((B, H), jnp.float32)
        hf_s[...] = z
        cf_s[...] = z
        hb_s[...] = z
        cb_s[...] = z

    def body(j, _):
        h, c = _lstm_step(pf_ref[j], hf_s[...], cf_s[...], whh_ref[0])
        hf_s[...] = h
        cf_s[...] = c
        Hf_s[pl.ds(j, 1)] = h.reshape(1, B, H)
        jr = C - 1 - j
        h2, c2 = _lstm_step(pb_ref[jr], hb_s[...], cb_s[...], whh_ref[1])
        valid = (tc > 0) | (jr < C - (TP - TC))
        h2 = jnp.where(valid, h2, hb_s[...])
        c2 = jnp.where(valid, c2, cb_s[...])
        hb_s[...] = h2
        cb_s[...] = c2
        Hb_s[pl.ds(jr, 1)] = h2.reshape(1, B, H)
        return ()

    jax.lax.fori_loop(0, C, body, ())
    sf = jnp.sum(Hf_s[...] * q_ref[0][None], axis=2)
    sf_ref[...] = sf[:, None, :]
    sb = jnp.sum(Hb_s[...] * q_ref[1][None], axis=2)
    sb_ref[...] = sb[:, None, :]


def _final_kernel(sf_ref, sb_ref, fcw_ref, fcb_ref, o_ref):
    s = sf_ref[:, 0, :] + sb_ref[:, 0, :]
    t_idx = jax.lax.broadcasted_iota(jnp.int32, (TP, B), 0)
    s = jnp.where(t_idx < TC, s, -1e30)
    m = jnp.max(s, axis=0, keepdims=True)
    e = jnp.exp(s - m)
    a = e / jnp.sum(e, axis=0, keepdims=True)
    logits = jnp.dot(fcw_ref[...], a, preferred_element_type=jnp.float32)
    logits = logits + fcb_ref[...]
    mx = jnp.max(logits, axis=0, keepdims=True)
    lse = jnp.log(jnp.sum(jnp.exp(logits - mx), axis=0, keepdims=True))
    o_ref[...] = (logits - mx - lse).T


def kernel(content, title, embed_w, t_wih_f, t_whh_f, t_b_f, t_wih_b,
           t_whh_b, t_b_b, c_wih_f, c_whh_f, c_b_f, c_wih_b, c_whh_b,
           c_b_b, att_w, fc_w, fc_b):
    f32 = jnp.float32
    bf16 = jnp.bfloat16
    idx = jnp.concatenate([content.T.reshape(-1),
                           jnp.zeros(N_ROWS - N_REAL, content.dtype)])
    half = N_ROWS // 2
    embed_i = jax.lax.bitcast_convert_type(
        embed_w.astype(bf16).reshape(-1, E // 2, 2), jnp.int32)
    xc_a = jax.lax.bitcast_convert_type(
        embed_i[idx[:half]], bf16).reshape(half, E)
    xc_b = jax.lax.bitcast_convert_type(
        embed_i[idx[half:]], bf16).reshape(half, E)
    xt = embed_w[title.T.reshape(-1)]
    cw2 = jnp.stack([c_wih_f.T.astype(bf16), c_wih_b.T.astype(bf16)])
    cb2 = jnp.stack([c_b_f, c_b_b]).reshape(2, 1, 4 * H)
    tw = jnp.concatenate([t_wih_f.T, t_wih_b.T], axis=1)
    tb = jnp.concatenate([t_b_f, t_b_b]).reshape(1, 8 * H)
    whh2 = jnp.stack([c_whh_f.T, c_whh_b.T]).astype(bf16)
    fcw_pad = jnp.concatenate([fc_w, jnp.zeros((5, TP - TC), f32)], axis=1)

    grid_h = (N_ROWS // 2) // PROJ_BM
    pre_shape = jax.ShapeDtypeStruct((TP, B, 4 * H), bf16)
    pre_f, pre_b = pl.pallas_call(
        _proj_kernel,
        grid=(grid_h,),
        in_specs=[
            pl.BlockSpec((PROJ_BM, E), lambda i: (i, 0)),
            pl.BlockSpec((2, E, 4 * H), lambda i: (0, 0, 0)),
            pl.BlockSpec((2, 1, 4 * H), lambda i: (0, 0, 0)),
        ],
        out_specs=[
            pl.BlockSpec((PROJ_BM // B, B, 4 * H), lambda i: (i, 0, 0)),
            pl.BlockSpec((PROJ_BM // B, B, 4 * H), lambda i: (i, 0, 0)),
        ],
        out_shape=[pre_shape, pre_shape],
        compiler_params=pltpu.CompilerParams(
            dimension_semantics=("arbitrary",),
            vmem_limit_bytes=40 * 1024 * 1024,
        ),
        name="content_proj",
    )(xc_a, cw2, cb2)
    pre_f, pre_b = pl.pallas_call(
        _proj2_kernel,
        grid=(grid_h,),
        in_specs=[
            pl.BlockSpec((PROJ_BM, E), lambda i: (i, 0)),
            pl.BlockSpec(memory_space=pl.ANY),
            pl.BlockSpec(memory_space=pl.ANY),
            pl.BlockSpec((2, E, 4 * H), lambda i: (0, 0, 0)),
            pl.BlockSpec((2, 1, 4 * H), lambda i: (0, 0, 0)),
        ],
        out_specs=[
            pl.BlockSpec((PROJ_BM // B, B, 4 * H),
                         lambda i: (i + grid_h, 0, 0)),
            pl.BlockSpec((PROJ_BM // B, B, 4 * H),
                         lambda i: (i + grid_h, 0, 0)),
        ],
        out_shape=[pre_shape, pre_shape],
        input_output_aliases={1: 0, 2: 1},
        compiler_params=pltpu.CompilerParams(
            dimension_semantics=("arbitrary",),
            vmem_limit_bytes=40 * 1024 * 1024,
        ),
        name="content_proj_b",
    )(xc_b, pre_f, pre_b, cw2, cb2)

    q = pl.pallas_call(
        _title_kernel,
        out_shape=jax.ShapeDtypeStruct((B, 2 * H), f32),
        scratch_shapes=[
            pltpu.VMEM((TT, B, 4 * H), f32),
            pltpu.VMEM((B, H), f32),
            pltpu.VMEM((B, H), f32),
        ],
        compiler_params=pltpu.CompilerParams(
            vmem_limit_bytes=48 * 1024 * 1024,
        ),
        name="title_q",
    )(xt, tw, tb, t_whh_f.T, t_whh_b.T, att_w)
    q2 = jnp.stack([q[:, :H], q[:, H:]])

    s_shape = jax.ShapeDtypeStruct((TP, 1, B), f32)
    sf, sb = pl.pallas_call(
        _scan_kernel,
        grid=(NC,),
        in_specs=[
            pl.BlockSpec((C, B, 4 * H), lambda i: (i, 0, 0)),
            pl.BlockSpec((C, B, 4 * H), lambda i: (NC - 1 - i, 0, 0)),
            pl.BlockSpec((2, H, 4 * H), lambda i: (0, 0, 0)),
            pl.BlockSpec((2, B, H), lambda i: (0, 0, 0)),
        ],
        out_specs=[
            pl.BlockSpec((C, 1, B), lambda i: (i, 0, 0)),
            pl.BlockSpec((C, 1, B), lambda i: (NC - 1 - i, 0, 0)),
        ],
        out_shape=[s_shape, s_shape],
        scratch_shapes=[
            pltpu.VMEM((B, H), f32),
            pltpu.VMEM((B, H), f32),
            pltpu.VMEM((B, H), f32),
            pltpu.VMEM((B, H), f32),
            pltpu.VMEM((C, B, H), f32),
            pltpu.VMEM((C, B, H), f32),
        ],
        compiler_params=pltpu.CompilerParams(
            dimension_semantics=("arbitrary",),
            vmem_limit_bytes=48 * 1024 * 1024,
        ),
        name="content_scan",
    )(pre_f, pre_b, whh2, q2)

    out = pl.pallas_call(
        _final_kernel,
        out_shape=jax.ShapeDtypeStruct((B, 5), f32),
        name="final",
    )(sf, sb, fcw_pad, fc_b.reshape(5, 1))
    return out


# speedup vs baseline: 2.4222x; 2.4222x over previous
import jax
import jax.numpy as jnp
from jax.experimental import pallas as pl
from jax.experimental.pallas import tpu as pltpu

B = 64
E = 300
H = 256
TC = 637
TT = 24
TP = 640
C = 16
NC = TP // C
PROJ_BM = 2048
N_ROWS = TP * B
N_REAL = TC * B


def _sigmoid(x):
    return jax.nn.sigmoid(x)


def _proj_kernel(x_ref, w_ref, b_ref, of_ref, ob_ref):
    x = x_ref[...].astype(jnp.bfloat16)
    rf = jnp.dot(x, w_ref[0], preferred_element_type=jnp.float32)
    of_ref[...] = (rf + b_ref[0]).astype(jnp.bfloat16).reshape(
        PROJ_BM // B, B, 4 * H)
    rb = jnp.dot(x, w_ref[1], preferred_element_type=jnp.float32)
    ob_ref[...] = (rb + b_ref[1]).astype(jnp.bfloat16).reshape(
        PROJ_BM // B, B, 4 * H)


def _proj2_kernel(x_ref, pfa_ref, pba_ref, w_ref, b_ref, of_ref, ob_ref):
    del pfa_ref, pba_ref
    _proj_kernel(x_ref, w_ref, b_ref, of_ref, ob_ref)


def _lstm_step(p, h, c, whh_t):
    g = p + jnp.dot(h.astype(whh_t.dtype), whh_t,
                    preferred_element_type=jnp.float32)
    i = g[:, :H]
    f = g[:, H:2 * H]
    gg = g[:, 2 * H:3 * H]
    o = g[:, 3 * H:]
    c_new = _sigmoid(f) * c + _sigmoid(i) * jnp.tanh(gg)
    h_new = _sigmoid(o) * jnp.tanh(c_new)
    return h_new, c_new


def _title_kernel(tx_ref, tw_ref, tb_ref, whf_ref, whb_ref, attw_ref, q_ref,
                  pre_s, h_s, c_s):
    tx = tx_ref[...]
    pf = jnp.dot(tx, tw_ref[:, :4 * H], preferred_element_type=jnp.float32)
    pre_s[...] = (pf + tb_ref[:, :4 * H]).reshape(TT, B, 4 * H)
    h_s[...] = jnp.zeros((B, H), jnp.float32)
    c_s[...] = jnp.zeros((B, H), jnp.float32)

    def body(j, _):
        h, c = _lstm_step(pre_s[j], h_s[...], c_s[...], whf_ref[...])
        h_s[...] = h
        c_s[...] = c
        return ()

    jax.lax.fori_loop(0, TT, body, ())
    pb = jnp.dot(tx[(TT - 1) * B:, :], tw_ref[:, 4 * H:],
                 preferred_element_type=jnp.float32) + tb_ref[:, 4 * H:]
    zero = jnp.zeros((B, H), jnp.float32)
    hb, _ = _lstm_step(pb, zero, zero, whb_ref[...])
    title_last = jnp.concatenate([h_s[...], hb], axis=1)
    q_ref[...] = jnp.dot(title_last, attw_ref[...],
                         preferred_element_type=jnp.float32)


def _scan_kernel(pf_ref, pb_ref, whh_ref, q_ref, sf_ref, sb_ref,
                 hf_s, cf_s, hb_s, cb_s, Hf_s, Hb_s):
    tc = pl.program_id(0)

    @pl.when(tc == 0)
    def _():
        z = jnp.zeros((B, H), jnp.float32)
        hf_s[...] = z
        cf_s[...] = z
        hb_s[...] = z
        cb_s[...] = z

    def body(j, _):
        h, c = _lstm_step(pf_ref[j], hf_s[...], cf_s[...], whh_ref[0])
        hf_s[...] = h
        cf_s[...] = c
        Hf_s[pl.ds(j, 1)] = h.reshape(1, B, H)
        jr = C - 1 - j
        h2, c2 = _lstm_step(pb_ref[jr], hb_s[...], cb_s[...], whh_ref[1])
        valid = (tc > 0) | (jr < C - (TP - TC))
        h2 = jnp.where(valid, h2, hb_s[...])
        c2 = jnp.where(valid, c2, cb_s[...])
        hb_s[...] = h2
        cb_s[...] = c2
        Hb_s[pl.ds(jr, 1)] = h2.reshape(1, B, H)
        return ()

    jax.lax.fori_loop(0, C, body, ())
    sf = jnp.sum(Hf_s[...] * q_ref[0][None], axis=2)
    sf_ref[...] = sf[:, None, :]
    sb = jnp.sum(Hb_s[...] * q_ref[1][None], axis=2)
    sb_ref[...] = sb[:, None, :]


def _final_kernel(sf_ref, sb_ref, fcw_ref, fcb_ref, o_ref):
    s = sf_ref[:, 0, :] + sb_ref[:, 0, :]
    t_idx = jax.lax.broadcasted_iota(jnp.int32, (TP, B), 0)
    s = jnp.where(t_idx < TC, s, -1e30)
    m = jnp.max(s, axis=0, keepdims=True)
    e = jnp.exp(s - m)
    a = e / jnp.sum(e, axis=0, keepdims=True)
    logits = jnp.dot(fcw_ref[...], a, preferred_element_type=jnp.float32)
    logits = logits + fcb_ref[...]
    mx = jnp.max(logits, axis=0, keepdims=True)
    lse = jnp.log(jnp.sum(jnp.exp(logits - mx), axis=0, keepdims=True))
    o_ref[...] = (logits - mx - lse).T


def kernel(content, title, embed_w, t_wih_f, t_whh_f, t_b_f, t_wih_b,
           t_whh_b, t_b_b, c_wih_f, c_whh_f, c_b_f, c_wih_b, c_whh_b,
           c_b_b, att_w, fc_w, fc_b):
    f32 = jnp.float32
    bf16 = jnp.bfloat16
    idx = jnp.concatenate([content.T.reshape(-1),
                           jnp.zeros(N_ROWS - N_REAL, content.dtype)])
    half = N_ROWS // 2
    xc_a = embed_w[idx[:half]]
    xc_b = embed_w[idx[half:]]
    xt = embed_w[title.T.reshape(-1)]
    cw2 = jnp.stack([c_wih_f.T.astype(bf16), c_wih_b.T.astype(bf16)])
    cb2 = jnp.stack([c_b_f, c_b_b]).reshape(2, 1, 4 * H)
    tw = jnp.concatenate([t_wih_f.T, t_wih_b.T], axis=1)
    tb = jnp.concatenate([t_b_f, t_b_b]).reshape(1, 8 * H)
    whh2 = jnp.stack([c_whh_f.T, c_whh_b.T]).astype(bf16)
    fcw_pad = jnp.concatenate([fc_w, jnp.zeros((5, TP - TC), f32)], axis=1)

    grid_h = (N_ROWS // 2) // PROJ_BM
    pre_shape = jax.ShapeDtypeStruct((TP, B, 4 * H), bf16)
    pre_f, pre_b = pl.pallas_call(
        _proj_kernel,
        grid=(grid_h,),
        in_specs=[
            pl.BlockSpec((PROJ_BM, E), lambda i: (i, 0)),
            pl.BlockSpec((2, E, 4 * H), lambda i: (0, 0, 0)),
            pl.BlockSpec((2, 1, 4 * H), lambda i: (0, 0, 0)),
        ],
        out_specs=[
            pl.BlockSpec((PROJ_BM // B, B, 4 * H), lambda i: (i, 0, 0)),
            pl.BlockSpec((PROJ_BM // B, B, 4 * H), lambda i: (i, 0, 0)),
        ],
        out_shape=[pre_shape, pre_shape],
        compiler_params=pltpu.CompilerParams(
            dimension_semantics=("arbitrary",),
            vmem_limit_bytes=40 * 1024 * 1024,
        ),
        name="content_proj",
    )(xc_a, cw2, cb2)
    pre_f, pre_b = pl.pallas_call(
        _proj2_kernel,
        grid=(grid_h,),
        in_specs=[
            pl.BlockSpec((PROJ_BM, E), lambda i: (i, 0)),
            pl.BlockSpec(memory_space=pl.ANY),
            pl.BlockSpec(memory_space=pl.ANY),
            pl.BlockSpec((2, E, 4 * H), lambda i: (0, 0, 0)),
            pl.BlockSpec((2, 1, 4 * H), lambda i: (0, 0, 0)),
        ],
        out_specs=[
            pl.BlockSpec((PROJ_BM // B, B, 4 * H),
                         lambda i: (i + grid_h, 0, 0)),
            pl.BlockSpec((PROJ_BM // B, B, 4 * H),
                         lambda i: (i + grid_h, 0, 0)),
        ],
        out_shape=[pre_shape, pre_shape],
        input_output_aliases={1: 0, 2: 1},
        compiler_params=pltpu.CompilerParams(
            dimension_semantics=("arbitrary",),
            vmem_limit_bytes=40 * 1024 * 1024,
        ),
        name="content_proj_b",
    )(xc_b, pre_f, pre_b, cw2, cb2)

    q = pl.pallas_call(
        _title_kernel,
        out_shape=jax.ShapeDtypeStruct((B, 2 * H), f32),
        scratch_shapes=[
            pltpu.VMEM((TT, B, 4 * H), f32),
            pltpu.VMEM((B, H), f32),
            pltpu.VMEM((B, H), f32),
        ],
        compiler_params=pltpu.CompilerParams(
            vmem_limit_bytes=48 * 1024 * 1024,
        ),
        name="title_q",
    )(xt, tw, tb, t_whh_f.T, t_whh_b.T, att_w)
    q2 = jnp.stack([q[:, :H], q[:, H:]])

    s_shape = jax.ShapeDtypeStruct((TP, 1, B), f32)
    sf, sb = pl.pallas_call(
        _scan_kernel,
        grid=(NC,),
        in_specs=[
            pl.BlockSpec((C, B, 4 * H), lambda i: (i, 0, 0)),
            pl.BlockSpec((C, B, 4 * H), lambda i: (NC - 1 - i, 0, 0)),
            pl.BlockSpec((2, H, 4 * H), lambda i: (0, 0, 0)),
            pl.BlockSpec((2, B, H), lambda i: (0, 0, 0)),
        ],
        out_specs=[
            pl.BlockSpec((C, 1, B), lambda i: (i, 0, 0)),
            pl.BlockSpec((C, 1, B), lambda i: (NC - 1 - i, 0, 0)),
        ],
        out_shape=[s_shape, s_shape],
        scratch_shapes=[
            pltpu.VMEM((B, H), f32),
            pltpu.VMEM((B, H), f32),
            pltpu.VMEM((B, H), f32),
            pltpu.VMEM((B, H), f32),
            pltpu.VMEM((C, B, H), f32),
            pltpu.VMEM((C, B, H), f32),
        ],
        compiler_params=pltpu.CompilerParams(
            dimension_semantics=("arbitrary",),
            vmem_limit_bytes=48 * 1024 * 1024,
        ),
        name="content_scan",
    )(pre_f, pre_b, whh2, q2)

    out = pl.pallas_call(
        _final_kernel,
        out_shape=jax.ShapeDtypeStruct((B, 5), f32),
        name="final",
    )(sf, sb, fcw_pad, fc_b.reshape(5, 1))
    return out


# speedup vs baseline: 2.5798x; 1.0651x over previous
import jax
import jax.numpy as jnp
from jax.experimental import pallas as pl
from jax.experimental.pallas import tpu as pltpu

B = 64
E = 300
H = 256
TC = 637
TT = 24
TP = 640
C = 16
NC = TP // C
PROJ_BM = 2048
N_ROWS = TP * B
N_REAL = TC * B


def _sigmoid(x):
    return jax.nn.sigmoid(x)


def _proj_kernel(x_ref, w_ref, b_ref, of_ref, ob_ref):
    x = x_ref[...].astype(jnp.bfloat16)
    rf = jnp.dot(x, w_ref[0], preferred_element_type=jnp.float32)
    of_ref[...] = (rf + b_ref[0]).astype(jnp.bfloat16).reshape(
        PROJ_BM // B, B, 4 * H)
    rb = jnp.dot(x, w_ref[1], preferred_element_type=jnp.float32)
    ob_ref[...] = (rb + b_ref[1]).astype(jnp.bfloat16).reshape(
        PROJ_BM // B, B, 4 * H)


def _proj2_kernel(x_ref, pfa_ref, pba_ref, w_ref, b_ref, of_ref, ob_ref):
    del pfa_ref, pba_ref
    _proj_kernel(x_ref, w_ref, b_ref, of_ref, ob_ref)


def _lstm_step(p, h, c, whh_t):
    g = p + jnp.dot(h.astype(whh_t.dtype), whh_t,
                    preferred_element_type=jnp.float32)
    i = g[:, :H]
    f = g[:, H:2 * H]
    gg = g[:, 2 * H:3 * H]
    o = g[:, 3 * H:]
    c_new = _sigmoid(f) * c + _sigmoid(i) * jnp.tanh(gg)
    h_new = _sigmoid(o) * jnp.tanh(c_new)
    return h_new, c_new


def _title_kernel(tx_ref, tw_ref, tb_ref, whf_ref, whb_ref, attw_ref, q_ref,
                  pre_s, h_s, c_s):
    tx = tx_ref[...]
    pf = jnp.dot(tx, tw_ref[:, :4 * H], preferred_element_type=jnp.float32)
    pre_s[...] = (pf + tb_ref[:, :4 * H]).reshape(TT, B, 4 * H)
    h_s[...] = jnp.zeros((B, H), jnp.float32)
    c_s[...] = jnp.zeros((B, H), jnp.float32)

    def body(j, _):
        h, c = _lstm_step(pre_s[j], h_s[...], c_s[...], whf_ref[...])
        h_s[...] = h
        c_s[...] = c
        return ()

    jax.lax.fori_loop(0, TT, body, ())
    pb = jnp.dot(tx[(TT - 1) * B:, :], tw_ref[:, 4 * H:],
                 preferred_element_type=jnp.float32) + tb_ref[:, 4 * H:]
    zero = jnp.zeros((B, H), jnp.float32)
    hb, _ = _lstm_step(pb, zero, zero, whb_ref[...])
    title_last = jnp.concatenate([h_s[...], hb], axis=1)
    q_ref[...] = jnp.dot(title_last, attw_ref[...],
                         preferred_element_type=jnp.float32)


def _scan_kernel(pf_ref, pb_ref, whh_ref, q_ref, sf_ref, sb_ref,
                 hf_s, cf_s, hb_s, cb_s, Hf_s, Hb_s):
    tc = pl.program_id(0)

    @pl.when(tc == 0)
    def _():
        z = jnp.zeros((B, H), jnp.float32)
        hf_s[...] = z
        cf_s[...] = z
        hb_s[...] = z
        cb_s[...] = z

    def body(j, _):
        h, c = _lstm_step(pf_ref[j], hf_s[...], cf_s[...], whh_ref[0])
        hf_s[...] = h
        cf_s[...] = c
        Hf_s[pl.ds(j, 1)] = h.reshape(1, B, H)
        jr = C - 1 - j
        h2, c2 = _lstm_step(pb_ref[jr], hb_s[...], cb_s[...], whh_ref[1])
        valid = (tc > 0) | (jr < C - (TP - TC))
        h2 = jnp.where(valid, h2, hb_s[...])
        c2 = jnp.where(valid, c2, cb_s[...])
        hb_s[...] = h2
        cb_s[...] = c2
        Hb_s[pl.ds(jr, 1)] = h2.reshape(1, B, H)
        return ()

    jax.lax.fori_loop(0, C, body, (), unroll=2)
    sf = jnp.sum(Hf_s[...] * q_ref[0][None], axis=2)
    sf_ref[...] = sf[:, None, :]
    sb = jnp.sum(Hb_s[...] * q_ref[1][None], axis=2)
    sb_ref[...] = sb[:, None, :]


def _final_kernel(sf_ref, sb_ref, fcw_ref, fcb_ref, o_ref):
    s = sf_ref[:, 0, :] + sb_ref[:, 0, :]
    t_idx = jax.lax.broadcasted_iota(jnp.int32, (TP, B), 0)
    s = jnp.where(t_idx < TC, s, -1e30)
    m = jnp.max(s, axis=0, keepdims=True)
    e = jnp.exp(s - m)
    a = e / jnp.sum(e, axis=0, keepdims=True)
    logits = jnp.dot(fcw_ref[...], a, preferred_element_type=jnp.float32)
    logits = logits + fcb_ref[...]
    mx = jnp.max(logits, axis=0, keepdims=True)
    lse = jnp.log(jnp.sum(jnp.exp(logits - mx), axis=0, keepdims=True))
    o_ref[...] = (logits - mx - lse).T


def kernel(content, title, embed_w, t_wih_f, t_whh_f, t_b_f, t_wih_b,
           t_whh_b, t_b_b, c_wih_f, c_whh_f, c_b_f, c_wih_b, c_whh_b,
           c_b_b, att_w, fc_w, fc_b):
    f32 = jnp.float32
    bf16 = jnp.bfloat16
    idx = jnp.concatenate([content.T.reshape(-1),
                           jnp.zeros(N_ROWS - N_REAL, content.dtype)])
    half = N_ROWS // 2
    xc_a = embed_w[idx[:half]]
    xc_b = embed_w[idx[half:]]
    xt = embed_w[title.T.reshape(-1)]
    cw2 = jnp.stack([c_wih_f.T.astype(bf16), c_wih_b.T.astype(bf16)])
    cb2 = jnp.stack([c_b_f, c_b_b]).reshape(2, 1, 4 * H)
    tw = jnp.concatenate([t_wih_f.T, t_wih_b.T], axis=1)
    tb = jnp.concatenate([t_b_f, t_b_b]).reshape(1, 8 * H)
    whh2 = jnp.stack([c_whh_f.T, c_whh_b.T]).astype(bf16)
    fcw_pad = jnp.concatenate([fc_w, jnp.zeros((5, TP - TC), f32)], axis=1)

    grid_h = (N_ROWS // 2) // PROJ_BM
    pre_shape = jax.ShapeDtypeStruct((TP, B, 4 * H), bf16)
    pre_f, pre_b = pl.pallas_call(
        _proj_kernel,
        grid=(grid_h,),
        in_specs=[
            pl.BlockSpec((PROJ_BM, E), lambda i: (i, 0)),
            pl.BlockSpec((2, E, 4 * H), lambda i: (0, 0, 0)),
            pl.BlockSpec((2, 1, 4 * H), lambda i: (0, 0, 0)),
        ],
        out_specs=[
            pl.BlockSpec((PROJ_BM // B, B, 4 * H), lambda i: (i, 0, 0)),
            pl.BlockSpec((PROJ_BM // B, B, 4 * H), lambda i: (i, 0, 0)),
        ],
        out_shape=[pre_shape, pre_shape],
        compiler_params=pltpu.CompilerParams(
            dimension_semantics=("arbitrary",),
            vmem_limit_bytes=40 * 1024 * 1024,
        ),
        name="content_proj",
    )(xc_a, cw2, cb2)
    pre_f, pre_b = pl.pallas_call(
        _proj2_kernel,
        grid=(grid_h,),
        in_specs=[
            pl.BlockSpec((PROJ_BM, E), lambda i: (i, 0)),
            pl.BlockSpec(memory_space=pl.ANY),
            pl.BlockSpec(memory_space=pl.ANY),
            pl.BlockSpec((2, E, 4 * H), lambda i: (0, 0, 0)),
            pl.BlockSpec((2, 1, 4 * H), lambda i: (0, 0, 0)),
        ],
        out_specs=[
            pl.BlockSpec((PROJ_BM // B, B, 4 * H),
                         lambda i: (i + grid_h, 0, 0)),
            pl.BlockSpec((PROJ_BM // B, B, 4 * H),
                         lambda i: (i + grid_h, 0, 0)),
        ],
        out_shape=[pre_shape, pre_shape],
        input_output_aliases={1: 0, 2: 1},
        compiler_params=pltpu.CompilerParams(
            dimension_semantics=("arbitrary",),
            vmem_limit_bytes=40 * 1024 * 1024,
        ),
        name="content_proj_b",
    )(xc_b, pre_f, pre_b, cw2, cb2)

    q = pl.pallas_call(
        _title_kernel,
        out_shape=jax.ShapeDtypeStruct((B, 2 * H), f32),
        scratch_shapes=[
            pltpu.VMEM((TT, B, 4 * H), f32),
            pltpu.VMEM((B, H), f32),
            pltpu.VMEM((B, H), f32),
        ],
        compiler_params=pltpu.CompilerParams(
            vmem_limit_bytes=48 * 1024 * 1024,
        ),
        name="title_q",
    )(xt, tw, tb, t_whh_f.T, t_whh_b.T, att_w)
    q2 = jnp.stack([q[:, :H], q[:, H:]])

    s_shape = jax.ShapeDtypeStruct((TP, 1, B), f32)
    sf, sb = pl.pallas_call(
        _scan_kernel,
        grid=(NC,),
        in_specs=[
            pl.BlockSpec((C, B, 4 * H), lambda i: (i, 0, 0)),
            pl.BlockSpec((C, B, 4 * H), lambda i: (NC - 1 - i, 0, 0)),
            pl.BlockSpec((2, H, 4 * H), lambda i: (0, 0, 0)),
            pl.BlockSpec((2, B, H), lambda i: (0, 0, 0)),
        ],
        out_specs=[
            pl.BlockSpec((C, 1, B), lambda i: (i, 0, 0)),
            pl.BlockSpec((C, 1, B), lambda i: (NC - 1 - i, 0, 0)),
        ],
        out_shape=[s_shape, s_shape],
        scratch_shapes=[
            pltpu.VMEM((B, H), f32),
            pltpu.VMEM((B, H), f32),
            pltpu.VMEM((B, H), f32),
            pltpu.VMEM((B, H), f32),
            pltpu.VMEM((C, B, H), f32),
            pltpu.VMEM((C, B, H), f32),
        ],
        compiler_params=pltpu.CompilerParams(
            dimension_semantics=("arbitrary",),
            vmem_limit_bytes=48 * 1024 * 1024,
        ),
        name="content_scan",
    )(pre_f, pre_b, whh2, q2)

    out = pl.pallas_call(
        _final_kernel,
        out_shape=jax.ShapeDtypeStruct((B, 5), f32),
        name="final",
    )(sf, sb, fcw_pad, fc_b.reshape(5, 1))
    return out


# speedup vs baseline: 2.6807x; 1.0391x over previous
import jax
import jax.numpy as jnp
from jax.experimental import pallas as pl
from jax.experimental.pallas import tpu as pltpu

B = 64
E = 300
H = 256
TC = 637
TT = 24
TP = 640
C = 16
NC = TP // C
PROJ_BM = 2048
N_ROWS = TP * B
N_REAL = TC * B


def _sigmoid(x):
    return jax.nn.sigmoid(x)


def _proj_kernel(x_ref, w_ref, b_ref, of_ref, ob_ref):
    x = x_ref[...].astype(jnp.bfloat16)
    rf = jnp.dot(x, w_ref[0], preferred_element_type=jnp.float32)
    of_ref[...] = (rf + b_ref[0]).astype(jnp.bfloat16).reshape(
        PROJ_BM // B, B, 4 * H)
    rb = jnp.dot(x, w_ref[1], preferred_element_type=jnp.float32)
    ob_ref[...] = (rb + b_ref[1]).astype(jnp.bfloat16).reshape(
        PROJ_BM // B, B, 4 * H)


def _proj2_kernel(x_ref, pfa_ref, pba_ref, w_ref, b_ref, of_ref, ob_ref):
    del pfa_ref, pba_ref
    _proj_kernel(x_ref, w_ref, b_ref, of_ref, ob_ref)


def _lstm_step(p, h, c, whh_t):
    g = p + jnp.dot(h.astype(whh_t.dtype), whh_t,
                    preferred_element_type=jnp.float32)
    i = g[:, :H]
    f = g[:, H:2 * H]
    gg = g[:, 2 * H:3 * H]
    o = g[:, 3 * H:]
    c_new = _sigmoid(f) * c + _sigmoid(i) * jnp.tanh(gg)
    h_new = _sigmoid(o) * jnp.tanh(c_new)
    return h_new, c_new


def _title_kernel(tx_ref, tw_ref, tb_ref, whf_ref, whb_ref, attw_ref, q_ref,
                  pre_s, h_s, c_s):
    tx = tx_ref[...]
    pf = jnp.dot(tx, tw_ref[:, :4 * H], preferred_element_type=jnp.float32)
    pre_s[...] = (pf + tb_ref[:, :4 * H]).reshape(TT, B, 4 * H)
    h_s[...] = jnp.zeros((B, H), jnp.float32)
    c_s[...] = jnp.zeros((B, H), jnp.float32)

    def body(j, _):
        h, c = _lstm_step(pre_s[j], h_s[...], c_s[...], whf_ref[...])
        h_s[...] = h
        c_s[...] = c
        return ()

    jax.lax.fori_loop(0, TT, body, ())
    pb = jnp.dot(tx[(TT - 1) * B:, :], tw_ref[:, 4 * H:],
                 preferred_element_type=jnp.float32) + tb_ref[:, 4 * H:]
    zero = jnp.zeros((B, H), jnp.float32)
    hb, _ = _lstm_step(pb, zero, zero, whb_ref[...])
    title_last = jnp.concatenate([h_s[...], hb], axis=1)
    q_ref[...] = jnp.dot(title_last, attw_ref[...],
                         preferred_element_type=jnp.float32)


def _scan_kernel(pf_ref, pb_ref, whh_ref, q_ref, sf_ref, sb_ref,
                 hf_s, cf_s, hb_s, cb_s, Hf_s, Hb_s):
    tc = pl.program_id(0)

    @pl.when(tc == 0)
    def _():
        z = jnp.zeros((B, H), jnp.float32)
        hf_s[...] = z
        cf_s[...] = z
        hb_s[...] = z
        cb_s[...] = z

    def body(j, _):
        h, c = _lstm_step(pf_ref[j], hf_s[...], cf_s[...], whh_ref[0])
        hf_s[...] = h
        cf_s[...] = c
        Hf_s[pl.ds(j, 1)] = h.reshape(1, B, H)
        jr = C - 1 - j
        h2, c2 = _lstm_step(pb_ref[jr], hb_s[...], cb_s[...], whh_ref[1])
        valid = (tc > 0) | (jr < C - (TP - TC))
        h2 = jnp.where(valid, h2, hb_s[...])
        c2 = jnp.where(valid, c2, cb_s[...])
        hb_s[...] = h2
        cb_s[...] = c2
        Hb_s[pl.ds(jr, 1)] = h2.reshape(1, B, H)
        return ()

    jax.lax.fori_loop(0, C, body, (), unroll=4)
    sf = jnp.sum(Hf_s[...] * q_ref[0][None], axis=2)
    sf_ref[...] = sf[:, None, :]
    sb = jnp.sum(Hb_s[...] * q_ref[1][None], axis=2)
    sb_ref[...] = sb[:, None, :]


def _final_kernel(sf_ref, sb_ref, fcw_ref, fcb_ref, o_ref):
    s = sf_ref[:, 0, :] + sb_ref[:, 0, :]
    t_idx = jax.lax.broadcasted_iota(jnp.int32, (TP, B), 0)
    s = jnp.where(t_idx < TC, s, -1e30)
    m = jnp.max(s, axis=0, keepdims=True)
    e = jnp.exp(s - m)
    a = e / jnp.sum(e, axis=0, keepdims=True)
    logits = jnp.dot(fcw_ref[...], a, preferred_element_type=jnp.float32)
    logits = logits + fcb_ref[...]
    mx = jnp.max(logits, axis=0, keepdims=True)
    lse = jnp.log(jnp.sum(jnp.exp(logits - mx), axis=0, keepdims=True))
    o_ref[...] = (logits - mx - lse).T


def kernel(content, title, embed_w, t_wih_f, t_whh_f, t_b_f, t_wih_b,
           t_whh_b, t_b_b, c_wih_f, c_whh_f, c_b_f, c_wih_b, c_whh_b,
           c_b_b, att_w, fc_w, fc_b):
    f32 = jnp.float32
    bf16 = jnp.bfloat16
    idx = jnp.concatenate([content.T.reshape(-1),
                           jnp.zeros(N_ROWS - N_REAL, content.dtype)])
    half = N_ROWS // 2
    xc_a = embed_w[idx[:half]]
    xc_b = embed_w[idx[half:]]
    xt = embed_w[title.T.reshape(-1)]
    cw2 = jnp.stack([c_wih_f.T.astype(bf16), c_wih_b.T.astype(bf16)])
    cb2 = jnp.stack([c_b_f, c_b_b]).reshape(2, 1, 4 * H)
    tw = jnp.concatenate([t_wih_f.T, t_wih_b.T], axis=1)
    tb = jnp.concatenate([t_b_f, t_b_b]).reshape(1, 8 * H)
    whh2 = jnp.stack([c_whh_f.T, c_whh_b.T]).astype(bf16)
    fcw_pad = jnp.concatenate([fc_w, jnp.zeros((5, TP - TC), f32)], axis=1)

    grid_h = (N_ROWS // 2) // PROJ_BM
    pre_shape = jax.ShapeDtypeStruct((TP, B, 4 * H), bf16)
    pre_f, pre_b = pl.pallas_call(
        _proj_kernel,
        grid=(grid_h,),
        in_specs=[
            pl.BlockSpec((PROJ_BM, E), lambda i: (i, 0)),
            pl.BlockSpec((2, E, 4 * H), lambda i: (0, 0, 0)),
            pl.BlockSpec((2, 1, 4 * H), lambda i: (0, 0, 0)),
        ],
        out_specs=[
            pl.BlockSpec((PROJ_BM // B, B, 4 * H), lambda i: (i, 0, 0)),
            pl.BlockSpec((PROJ_BM // B, B, 4 * H), lambda i: (i, 0, 0)),
        ],
        out_shape=[pre_shape, pre_shape],
        compiler_params=pltpu.CompilerParams(
            dimension_semantics=("arbitrary",),
            vmem_limit_bytes=40 * 1024 * 1024,
        ),
        name="content_proj",
    )(xc_a, cw2, cb2)
    pre_f, pre_b = pl.pallas_call(
        _proj2_kernel,
        grid=(grid_h,),
        in_specs=[
            pl.BlockSpec((PROJ_BM, E), lambda i: (i, 0)),
            pl.BlockSpec(memory_space=pl.ANY),
            pl.BlockSpec(memory_space=pl.ANY),
            pl.BlockSpec((2, E, 4 * H), lambda i: (0, 0, 0)),
            pl.BlockSpec((2, 1, 4 * H), lambda i: (0, 0, 0)),
        ],
        out_specs=[
            pl.BlockSpec((PROJ_BM // B, B, 4 * H),
                         lambda i: (i + grid_h, 0, 0)),
            pl.BlockSpec((PROJ_BM // B, B, 4 * H),
                         lambda i: (i + grid_h, 0, 0)),
        ],
        out_shape=[pre_shape, pre_shape],
        input_output_aliases={1: 0, 2: 1},
        compiler_params=pltpu.CompilerParams(
            dimension_semantics=("arbitrary",),
            vmem_limit_bytes=40 * 1024 * 1024,
        ),
        name="content_proj_b",
    )(xc_b, pre_f, pre_b, cw2, cb2)

    q = pl.pallas_call(
        _title_kernel,
        out_shape=jax.ShapeDtypeStruct((B, 2 * H), f32),
        scratch_shapes=[
            pltpu.VMEM((TT, B, 4 * H), f32),
            pltpu.VMEM((B, H), f32),
            pltpu.VMEM((B, H), f32),
        ],
        compiler_params=pltpu.CompilerParams(
            vmem_limit_bytes=48 * 1024 * 1024,
        ),
        name="title_q",
    )(xt, tw, tb, t_whh_f.T, t_whh_b.T, att_w)
    q2 = jnp.stack([q[:, :H], q[:, H:]])

    s_shape = jax.ShapeDtypeStruct((TP, 1, B), f32)
    sf, sb = pl.pallas_call(
        _scan_kernel,
        grid=(NC,),
        in_specs=[
            pl.BlockSpec((C, B, 4 * H), lambda i: (i, 0, 0)),
            pl.BlockSpec((C, B, 4 * H), lambda i: (NC - 1 - i, 0, 0)),
            pl.BlockSpec((2, H, 4 * H), lambda i: (0, 0, 0)),
            pl.BlockSpec((2, B, H), lambda i: (0, 0, 0)),
        ],
        out_specs=[
            pl.BlockSpec((C, 1, B), lambda i: (i, 0, 0)),
            pl.BlockSpec((C, 1, B), lambda i: (NC - 1 - i, 0, 0)),
        ],
        out_shape=[s_shape, s_shape],
        scratch_shapes=[
            pltpu.VMEM((B, H), f32),
            pltpu.VMEM((B, H), f32),
            pltpu.VMEM((B, H), f32),
            pltpu.VMEM((B, H), f32),
            pltpu.VMEM((C, B, H), f32),
            pltpu.VMEM((C, B, H), f32),
        ],
        compiler_params=pltpu.CompilerParams(
            dimension_semantics=("arbitrary",),
            vmem_limit_bytes=48 * 1024 * 1024,
        ),
        name="content_scan",
    )(pre_f, pre_b, whh2, q2)

    out = pl.pallas_call(
        _final_kernel,
        out_shape=jax.ShapeDtypeStruct((B, 5), f32),
        name="final",
    )(sf, sb, fcw_pad, fc_b.reshape(5, 1))
    return out


# speedup vs baseline: 2.7419x; 1.0228x over previous
import jax
import jax.numpy as jnp
from jax.experimental import pallas as pl
from jax.experimental.pallas import tpu as pltpu

B = 64
E = 300
H = 256
TC = 637
TT = 24
TP = 640
C = 16
NC = TP // C
PROJ_BM = 2048
N_ROWS = TP * B
N_REAL = TC * B


def _sigmoid(x):
    return jax.nn.sigmoid(x)


def _proj_kernel(x_ref, w_ref, b_ref, of_ref, ob_ref):
    x = x_ref[...].astype(jnp.bfloat16)
    rf = jnp.dot(x, w_ref[0], preferred_element_type=jnp.float32)
    of_ref[...] = (rf + b_ref[0]).astype(jnp.bfloat16).reshape(
        PROJ_BM // B, B, 4 * H)
    rb = jnp.dot(x, w_ref[1], preferred_element_type=jnp.float32)
    ob_ref[...] = (rb + b_ref[1]).astype(jnp.bfloat16).reshape(
        PROJ_BM // B, B, 4 * H)


def _proj2_kernel(x_ref, pfa_ref, pba_ref, w_ref, b_ref, of_ref, ob_ref):
    del pfa_ref, pba_ref
    _proj_kernel(x_ref, w_ref, b_ref, of_ref, ob_ref)


def _lstm_step(p, h, c, whh_t):
    g = p + jnp.dot(h.astype(whh_t.dtype), whh_t,
                    preferred_element_type=jnp.float32)
    i = g[:, :H]
    f = g[:, H:2 * H]
    gg = g[:, 2 * H:3 * H]
    o = g[:, 3 * H:]
    c_new = _sigmoid(f) * c + _sigmoid(i) * jnp.tanh(gg)
    h_new = _sigmoid(o) * jnp.tanh(c_new)
    return h_new, c_new


def _title_kernel(tx_ref, tw_ref, tb_ref, whf_ref, whb_ref, attw_ref, q_ref,
                  pre_s, h_s, c_s):
    tx = tx_ref[...]
    pf = jnp.dot(tx, tw_ref[:, :4 * H], preferred_element_type=jnp.float32)
    pre_s[...] = (pf + tb_ref[:, :4 * H]).reshape(TT, B, 4 * H)
    h_s[...] = jnp.zeros((B, H), jnp.float32)
    c_s[...] = jnp.zeros((B, H), jnp.float32)

    def body(j, _):
        h, c = _lstm_step(pre_s[j], h_s[...], c_s[...], whf_ref[...])
        h_s[...] = h
        c_s[...] = c
        return ()

    jax.lax.fori_loop(0, TT, body, ())
    pb = jnp.dot(tx[(TT - 1) * B:, :], tw_ref[:, 4 * H:],
                 preferred_element_type=jnp.float32) + tb_ref[:, 4 * H:]
    zero = jnp.zeros((B, H), jnp.float32)
    hb, _ = _lstm_step(pb, zero, zero, whb_ref[...])
    title_last = jnp.concatenate([h_s[...], hb], axis=1)
    q_ref[...] = jnp.dot(title_last, attw_ref[...],
                         preferred_element_type=jnp.float32)


def _scan_kernel(pf_ref, pb_ref, whh_ref, q_ref, sf_ref, sb_ref,
                 hf_s, cf_s, hb_s, cb_s, Hf_s, Hb_s):
    tc = pl.program_id(0)

    @pl.when(tc == 0)
    def _():
        z = jnp.zeros((B, H), jnp.float32)
        hf_s[...] = z
        cf_s[...] = z
        hb_s[...] = z
        cb_s[...] = z

    def body(j, _):
        h, c = _lstm_step(pf_ref[j], hf_s[...], cf_s[...], whh_ref[0])
        hf_s[...] = h
        cf_s[...] = c
        Hf_s[pl.ds(j, 1)] = h.reshape(1, B, H)
        jr = C - 1 - j
        h2, c2 = _lstm_step(pb_ref[jr], hb_s[...], cb_s[...], whh_ref[1])
        valid = (tc > 0) | (jr < C - (TP - TC))
        h2 = jnp.where(valid, h2, hb_s[...])
        c2 = jnp.where(valid, c2, cb_s[...])
        hb_s[...] = h2
        cb_s[...] = c2
        Hb_s[pl.ds(jr, 1)] = h2.reshape(1, B, H)
        return ()

    jax.lax.fori_loop(0, C, body, (), unroll=8)
    sf = jnp.sum(Hf_s[...] * q_ref[0][None], axis=2)
    sf_ref[...] = sf[:, None, :]
    sb = jnp.sum(Hb_s[...] * q_ref[1][None], axis=2)
    sb_ref[...] = sb[:, None, :]


def _final_kernel(sf_ref, sb_ref, fcw_ref, fcb_ref, o_ref):
    s = sf_ref[:, 0, :] + sb_ref[:, 0, :]
    t_idx = jax.lax.broadcasted_iota(jnp.int32, (TP, B), 0)
    s = jnp.where(t_idx < TC, s, -1e30)
    m = jnp.max(s, axis=0, keepdims=True)
    e = jnp.exp(s - m)
    a = e / jnp.sum(e, axis=0, keepdims=True)
    logits = jnp.dot(fcw_ref[...], a, preferred_element_type=jnp.float32)
    logits = logits + fcb_ref[...]
    mx = jnp.max(logits, axis=0, keepdims=True)
    lse = jnp.log(jnp.sum(jnp.exp(logits - mx), axis=0, keepdims=True))
    o_ref[...] = (logits - mx - lse).T


def kernel(content, title, embed_w, t_wih_f, t_whh_f, t_b_f, t_wih_b,
           t_whh_b, t_b_b, c_wih_f, c_whh_f, c_b_f, c_wih_b, c_whh_b,
           c_b_b, att_w, fc_w, fc_b):
    f32 = jnp.float32
    bf16 = jnp.bfloat16
    idx = jnp.concatenate([content.T.reshape(-1),
                           jnp.zeros(N_ROWS - N_REAL, content.dtype)])
    half = N_ROWS // 2
    xc_a = embed_w[idx[:half]]
    xc_b = embed_w[idx[half:]]
    xt = embed_w[title.T.reshape(-1)]
    cw2 = jnp.stack([c_wih_f.T.astype(bf16), c_wih_b.T.astype(bf16)])
    cb2 = jnp.stack([c_b_f, c_b_b]).reshape(2, 1, 4 * H)
    tw = jnp.concatenate([t_wih_f.T, t_wih_b.T], axis=1)
    tb = jnp.concatenate([t_b_f, t_b_b]).reshape(1, 8 * H)
    whh2 = jnp.stack([c_whh_f.T, c_whh_b.T]).astype(bf16)
    fcw_pad = jnp.concatenate([fc_w, jnp.zeros((5, TP - TC), f32)], axis=1)

    grid_h = (N_ROWS // 2) // PROJ_BM
    pre_shape = jax.ShapeDtypeStruct((TP, B, 4 * H), bf16)
    pre_f, pre_b = pl.pallas_call(
        _proj_kernel,
        grid=(grid_h,),
        in_specs=[
            pl.BlockSpec((PROJ_BM, E), lambda i: (i, 0)),
            pl.BlockSpec((2, E, 4 * H), lambda i: (0, 0, 0)),
            pl.BlockSpec((2, 1, 4 * H), lambda i: (0, 0, 0)),
        ],
        out_specs=[
            pl.BlockSpec((PROJ_BM // B, B, 4 * H), lambda i: (i, 0, 0)),
            pl.BlockSpec((PROJ_BM // B, B, 4 * H), lambda i: (i, 0, 0)),
        ],
        out_shape=[pre_shape, pre_shape],
        compiler_params=pltpu.CompilerParams(
            dimension_semantics=("arbitrary",),
            vmem_limit_bytes=40 * 1024 * 1024,
        ),
        name="content_proj",
    )(xc_a, cw2, cb2)
    pre_f, pre_b = pl.pallas_call(
        _proj2_kernel,
        grid=(grid_h,),
        in_specs=[
            pl.BlockSpec((PROJ_BM, E), lambda i: (i, 0)),
            pl.BlockSpec(memory_space=pl.ANY),
            pl.BlockSpec(memory_space=pl.ANY),
            pl.BlockSpec((2, E, 4 * H), lambda i: (0, 0, 0)),
            pl.BlockSpec((2, 1, 4 * H), lambda i: (0, 0, 0)),
        ],
        out_specs=[
            pl.BlockSpec((PROJ_BM // B, B, 4 * H),
                         lambda i: (i + grid_h, 0, 0)),
            pl.BlockSpec((PROJ_BM // B, B, 4 * H),
                         lambda i: (i + grid_h, 0, 0)),
        ],
        out_shape=[pre_shape, pre_shape],
        input_output_aliases={1: 0, 2: 1},
        compiler_params=pltpu.CompilerParams(
            dimension_semantics=("arbitrary",),
            vmem_limit_bytes=40 * 1024 * 1024,
        ),
        name="content_proj_b",
    )(xc_b, pre_f, pre_b, cw2, cb2)

    q = pl.pallas_call(
        _title_kernel,
        out_shape=jax.ShapeDtypeStruct((B, 2 * H), f32),
        scratch_shapes=[
            pltpu.VMEM((TT, B, 4 * H), f32),
            pltpu.VMEM((B, H), f32),
            pltpu.VMEM((B, H), f32),
        ],
        compiler_params=pltpu.CompilerParams(
            vmem_limit_bytes=48 * 1024 * 1024,
        ),
        name="title_q",
    )(xt, tw, tb, t_whh_f.T, t_whh_b.T, att_w)
    q2 = jnp.stack([q[:, :H], q[:, H:]])

    s_shape = jax.ShapeDtypeStruct((TP, 1, B), f32)
    sf, sb = pl.pallas_call(
        _scan_kernel,
        grid=(NC,),
        in_specs=[
            pl.BlockSpec((C, B, 4 * H), lambda i: (i, 0, 0)),
            pl.BlockSpec((C, B, 4 * H), lambda i: (NC - 1 - i, 0, 0)),
            pl.BlockSpec((2, H, 4 * H), lambda i: (0, 0, 0)),
            pl.BlockSpec((2, B, H), lambda i: (0, 0, 0)),
        ],
        out_specs=[
            pl.BlockSpec((C, 1, B), lambda i: (i, 0, 0)),
            pl.BlockSpec((C, 1, B), lambda i: (NC - 1 - i, 0, 0)),
        ],
        out_shape=[s_shape, s_shape],
        scratch_shapes=[
            pltpu.VMEM((B, H), f32),
            pltpu.VMEM((B, H), f32),
            pltpu.VMEM((B, H), f32),
            pltpu.VMEM((B, H), f32),
            pltpu.VMEM((C, B, H), f32),
            pltpu.VMEM((C, B, H), f32),
        ],
        compiler_params=pltpu.CompilerParams(
            dimension_semantics=("arbitrary",),
            vmem_limit_bytes=48 * 1024 * 1024,
        ),
        name="content_scan",
    )(pre_f, pre_b, whh2, q2)

    out = pl.pallas_call(
        _final_kernel,
        out_shape=jax.ShapeDtypeStruct((B, 5), f32),
        name="final",
    )(sf, sb, fcw_pad, fc_b.reshape(5, 1))
    return out
